# baseline (speedup 1.0000x reference)
import functools

import jax
import jax.numpy as jnp
from jax import lax
from jax.experimental import pallas as pl
from jax.experimental.pallas import tpu as pltpu
from jax.experimental.pallas import tpu_sc as plsc

VOCAB = 100000
EMB = 64
BATCH = 1024
CTX = 20

NUM_CORES = 2
NUM_SUBCORES = 16
NUM_WORKERS = NUM_CORES * NUM_SUBCORES
BPW = BATCH // NUM_WORKERS
IPW = BPW * CTX

LANES = 16
VB = 2048


def _pooled_sc(idx_flat, emb_table):
    mesh = plsc.VectorSubcoreMesh(core_axis_name="c", subcore_axis_name="s")

    @functools.partial(
        pl.kernel,
        mesh=mesh,
        out_type=jax.ShapeDtypeStruct((BATCH, EMB), jnp.float32),
        scratch_types=[
            pltpu.VMEM((IPW,), jnp.int32),
            pltpu.VMEM((IPW, EMB), jnp.float32),
            pltpu.VMEM((BPW, EMB), jnp.float32),
            pltpu.SemaphoreType.DMA,
        ],
        compiler_params=pltpu.CompilerParams(use_tc_tiling_on_sc=False),
    )
    def k(table_hbm, idx_hbm, out_hbm, idx_v, rows_v, pooled_v, sem):
        wid = lax.axis_index("s") * NUM_CORES + lax.axis_index("c")
        pltpu.sync_copy(idx_hbm.at[pl.ds(wid * IPW, IPW)], idx_v)
        pltpu.async_copy(table_hbm.at[idx_v], rows_v, sem).wait()

        def batch_body(b, carry):
            base = b * CTX
            for d in range(EMB // LANES):
                acc = rows_v[base, pl.ds(d * LANES, LANES)]

                def row_body(j, a):
                    return a + rows_v[base + j, pl.ds(d * LANES, LANES)]

                acc = lax.fori_loop(1, CTX, row_body, acc)
                pooled_v[b, pl.ds(d * LANES, LANES)] = acc
            return carry

        lax.fori_loop(0, BPW, batch_body, 0)
        pltpu.sync_copy(pooled_v, out_hbm.at[pl.ds(wid * BPW, BPW)])

    return k(emb_table, idx_flat)


NSTEPS = 49
TAIL = VOCAB - (NSTEPS - 1) * VB
NBUF = 4


def _mm_body(p_ref, w_ref, o_hbm, acc, tail_acc, sems, tail_sem):
    i = pl.program_id(0)
    slot = lax.rem(i, NBUF)

    for k in range(NBUF):
        @pl.when(jnp.logical_and(i >= NBUF, slot == k))
        def _(k=k):
            pltpu.make_async_copy(
                acc.at[k],
                o_hbm.at[:, pl.ds((i - NBUF) * VB, VB)],
                sems.at[k],
            ).wait()

    @pl.when(i < NSTEPS - 1)
    def _():
        acc[slot] = lax.dot_general(
            p_ref[...],
            w_ref[...],
            dimension_numbers=(((1,), (1,)), ((), ())),
            preferred_element_type=jnp.float32,
        )

    for k in range(NBUF):
        @pl.when(jnp.logical_and(i < NSTEPS - 1, slot == k))
        def _(k=k):
            pltpu.make_async_copy(
                acc.at[k], o_hbm.at[:, pl.ds(i * VB, VB)], sems.at[k]
            ).start()

    @pl.when(i == NSTEPS - 1)
    def _():
        tail_acc[...] = lax.dot_general(
            p_ref[...],
            w_ref[pl.ds(0, TAIL), :],
            dimension_numbers=(((1,), (1,)), ((), ())),
            preferred_element_type=jnp.float32,
        )
        pltpu.make_async_copy(
            tail_acc,
            o_hbm.at[:, pl.ds((NSTEPS - 1) * VB, TAIL)],
            tail_sem,
        ).start()
        for j in range(NSTEPS - NBUF, NSTEPS - 1):
            pltpu.make_async_copy(
                acc.at[j % NBUF],
                o_hbm.at[:, pl.ds(j * VB, VB)],
                sems.at[j % NBUF],
            ).wait()
        pltpu.make_async_copy(
            tail_acc,
            o_hbm.at[:, pl.ds((NSTEPS - 1) * VB, TAIL)],
            tail_sem,
        ).wait()


def kernel(x, emb_table, W):
    idx_flat = x.reshape(-1).astype(jnp.int32)
    pooled = _pooled_sc(idx_flat, emb_table)
    out = pl.pallas_call(
        _mm_body,
        grid=(NSTEPS,),
        in_specs=[
            pl.BlockSpec((BATCH, EMB), lambda i: (0, 0)),
            pl.BlockSpec((VB, EMB), lambda i: (i, 0)),
        ],
        out_specs=pl.BlockSpec(memory_space=pl.ANY),
        out_shape=jax.ShapeDtypeStruct((BATCH, VOCAB), jnp.float32),
        scratch_shapes=[
            pltpu.VMEM((NBUF, BATCH, VB), jnp.float32),
            pltpu.VMEM((BATCH, TAIL), jnp.float32),
            pltpu.SemaphoreType.DMA((NBUF,)),
            pltpu.SemaphoreType.DMA,
        ],
        compiler_params=pltpu.CompilerParams(
            dimension_semantics=("arbitrary",),
        ),
    )(pooled, W)
    return out


# speedup vs baseline: 2.7763x; 2.7763x over previous
import functools

import jax
import jax.numpy as jnp
from jax import lax
from jax.experimental import pallas as pl
from jax.experimental.pallas import tpu as pltpu
from jax.experimental.pallas import tpu_sc as plsc

VOCAB = 100000
EMB = 64
BATCH = 1024
CTX = 20

NUM_CORES = 2
NUM_SUBCORES = 16
NUM_WORKERS = NUM_CORES * NUM_SUBCORES
BPW = BATCH // NUM_WORKERS
IPW = BPW * CTX

LANES = 16
VB = 2048


def _pooled_sc(idx_flat, emb_table):
    mesh = plsc.VectorSubcoreMesh(core_axis_name="c", subcore_axis_name="s")

    @functools.partial(
        pl.kernel,
        mesh=mesh,
        out_type=jax.ShapeDtypeStruct((BATCH, EMB), jnp.float32),
        scratch_types=[
            pltpu.VMEM((IPW,), jnp.int32),
            pltpu.VMEM((IPW, EMB), jnp.float32),
            pltpu.VMEM((BPW, EMB), jnp.float32),
            pltpu.SemaphoreType.DMA,
        ],
        compiler_params=pltpu.CompilerParams(use_tc_tiling_on_sc=False),
    )
    def k(table_hbm, idx_hbm, out_hbm, idx_v, rows_v, pooled_v, sem):
        wid = lax.axis_index("s") * NUM_CORES + lax.axis_index("c")
        pltpu.sync_copy(idx_hbm.at[pl.ds(wid * IPW, IPW)], idx_v)
        pltpu.async_copy(table_hbm.at[idx_v], rows_v, sem).wait()

        def batch_body(b, carry):
            base = b * CTX
            for d in range(EMB // LANES):
                acc = rows_v[base, pl.ds(d * LANES, LANES)]

                def row_body(j, a):
                    return a + rows_v[base + j, pl.ds(d * LANES, LANES)]

                acc = lax.fori_loop(1, CTX, row_body, acc)
                pooled_v[b, pl.ds(d * LANES, LANES)] = acc
            return carry

        lax.fori_loop(0, BPW, batch_body, 0)
        pltpu.sync_copy(pooled_v, out_hbm.at[pl.ds(wid * BPW, BPW)])

    return k(emb_table, idx_flat)


def _mm_body(p_ref, wt_ref, ot_ref):
    ot_ref[...] = lax.dot_general(
        wt_ref[...],
        p_ref[...],
        dimension_numbers=(((0,), (1,)), ((), ())),
        preferred_element_type=jnp.float32,
    )


def kernel(x, emb_table, W):
    idx_flat = x.reshape(-1).astype(jnp.int32)
    pooled = _pooled_sc(idx_flat, emb_table)
    wt = W.T
    out_t = pl.pallas_call(
        _mm_body,
        grid=(pl.cdiv(VOCAB, VB),),
        in_specs=[
            pl.BlockSpec((BATCH, EMB), lambda i: (0, 0)),
            pl.BlockSpec((EMB, VB), lambda i: (0, i)),
        ],
        out_specs=pl.BlockSpec((VB, BATCH), lambda i: (i, 0)),
        out_shape=jax.ShapeDtypeStruct((VOCAB, BATCH), jnp.float32),
        compiler_params=pltpu.CompilerParams(
            dimension_semantics=("arbitrary",),
        ),
    )(pooled, wt)
    return out_t.T
